# Initial kernel scaffold; baseline (speedup 1.0000x reference)
import jax
import jax.numpy as jnp
from jax.experimental import pallas as pl


def kernel(x, input_embedding):
    raise NotImplementedError("write your pallas kernel here")


# speedup vs baseline: 3.1479x; 3.1479x over previous
import functools
import math

import jax
import jax.numpy as jnp
import numpy as np
from jax import lax
from jax.experimental import pallas as pl
from jax.experimental.pallas import tpu as pltpu
from jax.experimental.pallas import tpu_sc as plsc

VOCAB_SIZE = 100000
EMBED_DIM = 128
BATCH = 4096
SEQ = 50

N_ROWS = BATCH * SEQ
NUM_CORES = 2
NUM_SUBCORES = 16
NUM_WORKERS = NUM_CORES * NUM_SUBCORES
ROWS_PER_WORKER = N_ROWS // NUM_WORKERS
CHUNK = 128
NUM_CHUNKS = ROWS_PER_WORKER // CHUNK

SCALE = float(np.float32(np.sqrt(np.float32(EMBED_DIM))))

_mesh = plsc.VectorSubcoreMesh(core_axis_name="c", subcore_axis_name="s")


@functools.partial(
    pl.kernel,
    mesh=_mesh,
    out_type=jax.ShapeDtypeStruct((N_ROWS, EMBED_DIM), jnp.float32),
    scratch_types=[
        pltpu.VMEM((NUM_CHUNKS, CHUNK), jnp.int32),
        pltpu.VMEM((2, CHUNK, EMBED_DIM), jnp.float32),
        pltpu.SemaphoreType.DMA,
    ],
)
def _embed_lookup(x_hbm, tab_hbm, out_hbm, idx_v, rows_v, gsem):
    wid = lax.axis_index("s") * NUM_CORES + lax.axis_index("c")
    base = wid * ROWS_PER_WORKER

    pltpu.sync_copy(x_hbm.at[wid], idx_v)

    def start_gather(ci, buf):
        pltpu.async_copy(tab_hbm.at[idx_v.at[ci]], rows_v.at[buf], gsem)

    def wait_gather(ci, buf):
        pltpu.make_async_copy(tab_hbm.at[idx_v.at[ci]], rows_v.at[buf], gsem).wait()

    def scale_chunk(buf):
        def row_body(r, _):
            for j in range(EMBED_DIM // 16):
                sl = pl.ds(j * 16, 16)
                rows_v[buf, r, sl] = rows_v[buf, r, sl] * SCALE
            return _
        lax.fori_loop(0, CHUNK, row_body, None, unroll=2)

    start_gather(0, 0)

    def body(cc, _):
        for b in range(2):
            ci = cc + b
            wait_gather(ci, b)

            @pl.when(ci + 1 < NUM_CHUNKS)
            def _():
                start_gather(ci + 1, 1 - b)

            scale_chunk(b)
            pltpu.sync_copy(
                rows_v.at[b], out_hbm.at[pl.ds(base + ci * CHUNK, CHUNK)]
            )
        return _

    lax.fori_loop(0, NUM_CHUNKS // 2, lambda i, c: body(i * 2, c), None)


def kernel(x, input_embedding):
    xf = x.reshape(NUM_WORKERS, NUM_CHUNKS, CHUNK)
    out = _embed_lookup(xf, input_embedding)
    return out.reshape(BATCH, SEQ, EMBED_DIM)


# speedup vs baseline: 4.1452x; 1.3168x over previous
import functools

import jax
import jax.numpy as jnp
import numpy as np
from jax import lax
from jax.experimental import pallas as pl
from jax.experimental.pallas import tpu as pltpu
from jax.experimental.pallas import tpu_sc as plsc

VOCAB_SIZE = 100000
EMBED_DIM = 128
BATCH = 4096
SEQ = 50

NUM_CORES = 2
NUM_SUBCORES = 16
NUM_WORKERS = NUM_CORES * NUM_SUBCORES
BATCH_PER_WORKER = BATCH // NUM_WORKERS
SLOT = 56

SCALE = float(np.float32(np.sqrt(np.float32(EMBED_DIM))))

_mesh = plsc.VectorSubcoreMesh(core_axis_name="c", subcore_axis_name="s")


@functools.partial(
    pl.kernel,
    mesh=_mesh,
    out_type=jax.ShapeDtypeStruct((BATCH, SEQ, EMBED_DIM), jnp.float32),
    scratch_types=[
        pltpu.VMEM((BATCH_PER_WORKER, SEQ), jnp.int32),
        pltpu.VMEM((2 * SLOT, EMBED_DIM), jnp.float32),
        pltpu.SemaphoreType.DMA,
    ],
)
def _embed_lookup(x_hbm, tab_hbm, out_hbm, idx_v, rows_v, gsem):
    wid = lax.axis_index("s") * NUM_CORES + lax.axis_index("c")
    b0 = wid * BATCH_PER_WORKER

    pltpu.sync_copy(x_hbm.at[pl.ds(b0, BATCH_PER_WORKER)], idx_v)

    def start_gather(bi, slot):
        pltpu.async_copy(
            tab_hbm.at[idx_v.at[bi]], rows_v.at[pl.ds(slot * SLOT, SEQ)], gsem
        )

    def wait_gather(bi, slot):
        pltpu.make_async_copy(
            tab_hbm.at[idx_v.at[bi]], rows_v.at[pl.ds(slot * SLOT, SEQ)], gsem
        ).wait()

    def scale_slot(slot):
        def row_body(r, _):
            for j in range(EMBED_DIM // 16):
                sl = pl.ds(j * 16, 16)
                rows_v[slot * SLOT + r, sl] = rows_v[slot * SLOT + r, sl] * SCALE
            return _
        lax.fori_loop(0, SEQ, row_body, None, unroll=2)

    start_gather(0, 0)

    def body(bb, _):
        for s in range(2):
            bi = bb + s
            wait_gather(bi, s)

            @pl.when(bi + 1 < BATCH_PER_WORKER)
            def _():
                start_gather(bi + 1, 1 - s)

            scale_slot(s)
            pltpu.sync_copy(rows_v.at[pl.ds(s * SLOT, SEQ)], out_hbm.at[b0 + bi])
        return _

    lax.fori_loop(0, BATCH_PER_WORKER // 2, lambda i, c: body(i * 2, c), None)


def kernel(x, input_embedding):
    return _embed_lookup(x, input_embedding)


# speedup vs baseline: 5.1736x; 1.2481x over previous
import functools

import jax
import jax.numpy as jnp
import numpy as np
from jax import lax
from jax.experimental import pallas as pl
from jax.experimental.pallas import tpu as pltpu
from jax.experimental.pallas import tpu_sc as plsc

VOCAB_SIZE = 100000
EMBED_DIM = 128
BATCH = 4096
SEQ = 50

NUM_CORES = 2
NUM_SUBCORES = 16
NUM_WORKERS = NUM_CORES * NUM_SUBCORES
BATCH_PER_WORKER = BATCH // NUM_WORKERS
CHUNK = 2 * SEQ
NUM_CHUNKS = BATCH_PER_WORKER // 2
SLOT = 104

SCALE = float(np.float32(np.sqrt(np.float32(EMBED_DIM))))

_mesh = plsc.VectorSubcoreMesh(core_axis_name="c", subcore_axis_name="s")


@functools.partial(
    pl.kernel,
    mesh=_mesh,
    out_type=jax.ShapeDtypeStruct((BATCH, SEQ, EMBED_DIM), jnp.float32),
    scratch_types=[
        pltpu.VMEM((NUM_CHUNKS, CHUNK), jnp.int32),
        pltpu.VMEM((2 * SLOT, EMBED_DIM), jnp.float32),
        pltpu.SemaphoreType.DMA,
    ],
)
def _embed_lookup(x_hbm, tab_hbm, out_hbm, idx_v, rows_v, gsem):
    wid = lax.axis_index("s") * NUM_CORES + lax.axis_index("c")
    b0 = wid * BATCH_PER_WORKER

    pltpu.sync_copy(x_hbm.at[wid], idx_v)

    def start_gather(ci, slot):
        pltpu.async_copy(
            tab_hbm.at[idx_v.at[ci]], rows_v.at[pl.ds(slot * SLOT, CHUNK)], gsem
        )

    def wait_gather(ci, slot):
        pltpu.make_async_copy(
            tab_hbm.at[idx_v.at[ci]], rows_v.at[pl.ds(slot * SLOT, CHUNK)], gsem
        ).wait()

    def scale_slot(slot):
        def row_body(r, _):
            for j in range(EMBED_DIM // 16):
                sl = pl.ds(j * 16, 16)
                rows_v[slot * SLOT + r, sl] = rows_v[slot * SLOT + r, sl] * SCALE
            return _
        lax.fori_loop(0, CHUNK, row_body, None, unroll=2)

    start_gather(0, 0)

    def body(cc, _):
        for s in range(2):
            ci = cc + s
            wait_gather(ci, s)

            @pl.when(ci + 1 < NUM_CHUNKS)
            def _():
                start_gather(ci + 1, 1 - s)

            scale_slot(s)
            pltpu.sync_copy(
                rows_v.at[pl.ds(s * SLOT, SEQ)], out_hbm.at[b0 + 2 * ci]
            )
            pltpu.sync_copy(
                rows_v.at[pl.ds(s * SLOT + SEQ, SEQ)], out_hbm.at[b0 + 2 * ci + 1]
            )
        return _

    lax.fori_loop(0, NUM_CHUNKS // 2, lambda i, c: body(i * 2, c), None)


def kernel(x, input_embedding):
    xf = x.reshape(NUM_WORKERS, NUM_CHUNKS, CHUNK)
    return _embed_lookup(xf, input_embedding)


# speedup vs baseline: 8.9781x; 1.7354x over previous
import functools

import jax
import jax.numpy as jnp
import numpy as np
from jax import lax
from jax.experimental import pallas as pl
from jax.experimental.pallas import tpu as pltpu
from jax.experimental.pallas import tpu_sc as plsc

VOCAB_SIZE = 100000
EMBED_DIM = 128
BATCH = 4096
SEQ = 50

NUM_CORES = 2
NUM_SUBCORES = 16
NUM_WORKERS = NUM_CORES * NUM_SUBCORES
BATCH_PER_WORKER = BATCH // NUM_WORKERS
SLOT = BATCH_PER_WORKER

SCALE = float(np.float32(np.sqrt(np.float32(EMBED_DIM))))

_mesh = plsc.VectorSubcoreMesh(core_axis_name="c", subcore_axis_name="s")


@functools.partial(
    pl.kernel,
    mesh=_mesh,
    out_type=jax.ShapeDtypeStruct((SEQ, BATCH, EMBED_DIM), jnp.float32),
    scratch_types=[
        pltpu.VMEM((SEQ, BATCH_PER_WORKER), jnp.int32),
        pltpu.VMEM((2 * SLOT, EMBED_DIM), jnp.float32),
        pltpu.SemaphoreType.DMA,
    ],
)
def _embed_lookup(x_hbm, tab_hbm, out_hbm, idx_v, rows_v, gsem):
    wid = lax.axis_index("s") * NUM_CORES + lax.axis_index("c")
    b0 = wid * BATCH_PER_WORKER

    pltpu.sync_copy(x_hbm.at[wid], idx_v)

    def start_gather(si, slot):
        pltpu.async_copy(
            tab_hbm.at[idx_v.at[si]], rows_v.at[pl.ds(slot * SLOT, SLOT)], gsem
        )

    def wait_gather(si, slot):
        pltpu.make_async_copy(
            tab_hbm.at[idx_v.at[si]], rows_v.at[pl.ds(slot * SLOT, SLOT)], gsem
        ).wait()

    def scale_slot(slot):
        def row_body(r, _):
            for j in range(EMBED_DIM // 16):
                sl = pl.ds(j * 16, 16)
                rows_v[slot * SLOT + r, sl] = rows_v[slot * SLOT + r, sl] * SCALE
            return _
        lax.fori_loop(0, SLOT, row_body, None, unroll=2)

    start_gather(0, 0)

    def body(ss, _):
        for s in range(2):
            si = ss + s
            wait_gather(si, s)

            @pl.when(si + 1 < SEQ)
            def _():
                start_gather(si + 1, 1 - s)

            scale_slot(s)
            pltpu.sync_copy(
                rows_v.at[pl.ds(s * SLOT, SLOT)],
                out_hbm.at[si, pl.ds(b0, BATCH_PER_WORKER)],
            )
        return _

    lax.fori_loop(0, SEQ // 2, lambda i, c: body(i * 2, c), None)


def kernel(x, input_embedding):
    xprep = x.reshape(NUM_WORKERS, BATCH_PER_WORKER, SEQ).transpose(0, 2, 1)
    out_sm = _embed_lookup(xprep, input_embedding)
    return out_sm.transpose(1, 0, 2)


# speedup vs baseline: 10.1387x; 1.1293x over previous
import functools

import jax
import jax.numpy as jnp
import numpy as np
from jax import lax
from jax.experimental import pallas as pl
from jax.experimental.pallas import tpu as pltpu
from jax.experimental.pallas import tpu_sc as plsc

VOCAB_SIZE = 100000
EMBED_DIM = 128
BATCH = 4096
SEQ = 50

NUM_CORES = 2
NUM_SUBCORES = 16
NUM_WORKERS = NUM_CORES * NUM_SUBCORES
BATCH_PER_WORKER = BATCH // NUM_WORKERS
SLOT = BATCH_PER_WORKER
NBUF = 4

SCALE = float(np.float32(np.sqrt(np.float32(EMBED_DIM))))

_mesh = plsc.VectorSubcoreMesh(core_axis_name="c", subcore_axis_name="s")


@functools.partial(
    pl.kernel,
    mesh=_mesh,
    out_type=jax.ShapeDtypeStruct((SEQ, BATCH, EMBED_DIM), jnp.float32),
    scratch_types=[
        pltpu.VMEM((SEQ, BATCH_PER_WORKER), jnp.int32),
        pltpu.VMEM((NBUF * SLOT, EMBED_DIM), jnp.float32),
        [pltpu.SemaphoreType.DMA] * NBUF,
        [pltpu.SemaphoreType.DMA] * NBUF,
    ],
)
def _embed_lookup(x_hbm, tab_hbm, out_hbm, idx_v, rows_v, gsems, ssems):
    wid = lax.axis_index("s") * NUM_CORES + lax.axis_index("c")
    b0 = wid * BATCH_PER_WORKER

    pltpu.sync_copy(x_hbm.at[wid], idx_v)

    def gather_refs(si, slot):
        return tab_hbm.at[idx_v.at[si]], rows_v.at[pl.ds(slot * SLOT, SLOT)]

    def start_gather(si, slot):
        src, dst = gather_refs(si, slot)
        pltpu.async_copy(src, dst, gsems[slot])

    def wait_gather(si, slot):
        src, dst = gather_refs(si, slot)
        pltpu.make_async_copy(src, dst, gsems[slot]).wait()

    def store_refs(si, slot):
        return (
            rows_v.at[pl.ds(slot * SLOT, SLOT)],
            out_hbm.at[si, pl.ds(b0, BATCH_PER_WORKER)],
        )

    def start_store(si, slot):
        src, dst = store_refs(si, slot)
        pltpu.async_copy(src, dst, ssems[slot])

    def wait_store(si, slot):
        src, dst = store_refs(si, slot)
        pltpu.make_async_copy(src, dst, ssems[slot]).wait()

    def scale_slot(slot):
        def row_body(r, _):
            for j in range(EMBED_DIM // 16):
                sl = pl.ds(j * 16, 16)
                rows_v[slot * SLOT + r, sl] = rows_v[slot * SLOT + r, sl] * SCALE
            return _
        lax.fori_loop(0, SLOT, row_body, None, unroll=2)

    def step(si, slot):
        wait_gather(si, slot)
        scale_slot(slot)
        start_store(si, slot)

    start_gather(0, 0)
    start_gather(1, 1)
    step(0, 0)
    start_gather(2, 2)
    step(1, 1)
    start_gather(3, 3)
    step(2, 2)
    wait_store(0, 0)
    start_gather(4, 0)
    step(3, 3)
    wait_store(1, 1)
    start_gather(5, 1)

    def body(i, _):
        base = 4 + i * 4
        for s in range(NBUF):
            si = base + s
            wait_gather(si, s)
            scale_slot(s)
            start_store(si, s)
            nxt = (s + 2) % NBUF
            wait_store(si - 2, nxt)
            start_gather(si + 2, nxt)
        return _

    lax.fori_loop(0, (SEQ - 6) // NBUF, body, None)

    step(48, 0)
    step(49, 1)
    wait_store(46, 2)
    wait_store(47, 3)
    wait_store(48, 0)
    wait_store(49, 1)


def kernel(x, input_embedding):
    xprep = x.reshape(NUM_WORKERS, BATCH_PER_WORKER, SEQ).transpose(0, 2, 1)
    out_sm = _embed_lookup(xprep, input_embedding)
    return out_sm.transpose(1, 0, 2)
